# baseline (speedup 1.0000x reference)
import functools

import jax
import jax.numpy as jnp
from jax import lax
from jax.experimental import pallas as pl
from jax.experimental.pallas import tpu as pltpu
from jax.experimental.pallas import tpu_sc as plsc

_VOCAB = 100000
_OUT = 64
_B = 1024
_L = 200

_NC = 2
_NS = 16
_NW = _NC * _NS
_RPW = _B // _NW
_LANES = 16
_CHUNKS = _OUT // _LANES
_UNROLL = 8


def _bow_body(
    tok_hbm, w_hbm, b_hbm, out_hbm, idx_v, rows0, rows1, b_v, out_v, sem0, sem1
):
    wid = lax.axis_index("s") * _NC + lax.axis_index("c")
    base = wid * _RPW

    pltpu.sync_copy(tok_hbm.at[pl.ds(base, _RPW)], idx_v)
    pltpu.sync_copy(b_hbm, b_v)

    bias = [b_v[pl.ds(c * _LANES, _LANES)] for c in range(_CHUNKS)]

    def gather(r, buf, sem):
        return pltpu.make_async_copy(
            w_hbm.at[idx_v.at[r]], buf, sem
        )

    gather(0, rows0, sem0).start()
    gather(1, rows1, sem1).start()

    @pl.loop(0, _RPW, step=2)
    def _(r):
        for k, (buf, sem) in enumerate(((rows0, sem0), (rows1, sem1))):
            rr = r + k
            gather(rr, buf, sem).wait()

            def acc_body(j, accs, buf=buf):
                accs = list(accs)
                row = j * _UNROLL
                for u in range(_UNROLL):
                    for c in range(_CHUNKS):
                        a = (u % 2) * _CHUNKS + c
                        accs[a] = accs[a] + buf[row + u, pl.ds(c * _LANES, _LANES)]
                return tuple(accs)

            zero = jnp.zeros((_LANES,), jnp.float32)
            accs = lax.fori_loop(0, _L // _UNROLL, acc_body, (zero,) * (2 * _CHUNKS))
            accs = [accs[c] + accs[_CHUNKS + c] for c in range(_CHUNKS)]

            @pl.when(rr + 2 < _RPW)
            def _(buf=buf, sem=sem, rr=rr):
                gather(rr + 2, buf, sem).start()

            for c in range(_CHUNKS):
                out_v[rr, pl.ds(c * _LANES, _LANES)] = jnp.maximum(
                    accs[c] + bias[c], 0.0
                )

    pltpu.sync_copy(out_v, out_hbm.at[pl.ds(base, _RPW)])


_TCV = 512


def _linearize_body(wt_ref, out_ref):
    x = wt_ref[...]
    r = jax.lax.broadcasted_iota(jnp.int32, (_TCV // 2, _TCV), 0)
    v = jax.lax.broadcasted_iota(jnp.int32, (_TCV // 2, _TCV), 1)
    e_even = jnp.where(v == 2 * r, 1.0, 0.0).astype(jnp.float32)
    e_odd = jnp.where(v == 2 * r + 1, 1.0, 0.0).astype(jnp.float32)
    dn = (((1,), (1,)), ((), ()))
    o_left = jax.lax.dot_general(
        e_even, x, dn, precision=jax.lax.Precision.HIGHEST,
        preferred_element_type=jnp.float32)
    o_right = jax.lax.dot_general(
        e_odd, x, dn, precision=jax.lax.Precision.HIGHEST,
        preferred_element_type=jnp.float32)
    out_ref[...] = jnp.concatenate([o_left, o_right], axis=1)


def _linearize(Wt):
    grid = (_VOCAB + _TCV - 1) // _TCV
    return pl.pallas_call(
        _linearize_body,
        out_shape=jax.ShapeDtypeStruct((_VOCAB // 2, 2 * _OUT), jnp.float32),
        grid=(grid,),
        in_specs=[pl.BlockSpec((_OUT, _TCV), lambda i: (0, i))],
        out_specs=pl.BlockSpec((_TCV // 2, 2 * _OUT), lambda i: (i, 0)),
    )(Wt)


@jax.jit
def kernel(tokens, W, b):
    tok = tokens.astype(jnp.int32)
    w_lin = _linearize(W.T).reshape(_VOCAB, _OUT)
    run = functools.partial(
        pl.kernel,
        out_type=jax.ShapeDtypeStruct((_B, _OUT), jnp.float32),
        mesh=plsc.VectorSubcoreMesh(core_axis_name="c", subcore_axis_name="s"),
        scratch_types=[
            pltpu.VMEM((_RPW, _L), jnp.int32),
            pltpu.VMEM((_L, _OUT), jnp.float32),
            pltpu.VMEM((_L, _OUT), jnp.float32),
            pltpu.VMEM((_OUT,), jnp.float32),
            pltpu.VMEM((_RPW, _OUT), jnp.float32),
            pltpu.SemaphoreType.DMA,
            pltpu.SemaphoreType.DMA,
        ],
        compiler_params=pltpu.CompilerParams(use_tc_tiling_on_sc=False),
    )(_bow_body)
    return run(tok, w_lin, b)


# speedup vs baseline: 2.7412x; 2.7412x over previous
import functools

import jax
import jax.numpy as jnp
from jax import lax
from jax.experimental import pallas as pl
from jax.experimental.pallas import tpu as pltpu
from jax.experimental.pallas import tpu_sc as plsc

_VOCAB = 100000
_OUT = 64
_B = 1024
_L = 200

_NC = 2
_NS = 16
_NW = _NC * _NS
_RPW = _B // _NW
_LANES = 16
_CHUNKS = _OUT // _LANES
_UNROLL = 8


def _bow_body(
    tok_hbm, w_hbm, b_hbm, out_hbm, idx_v, rows0, rows1, b_v, out_v, sem0, sem1
):
    wid = lax.axis_index("s") * _NC + lax.axis_index("c")
    base = wid * _RPW

    pltpu.sync_copy(tok_hbm.at[pl.ds(base, _RPW)], idx_v)
    pltpu.sync_copy(b_hbm, b_v)

    bias = [b_v[pl.ds(c * _LANES, _LANES)] for c in range(_CHUNKS)]

    def gather(r, buf, sem):
        return pltpu.make_async_copy(
            w_hbm.at[idx_v.at[r]], buf, sem
        )

    gather(0, rows0, sem0).start()
    gather(1, rows1, sem1).start()

    @pl.loop(0, _RPW, step=2)
    def _(r):
        for k, (buf, sem) in enumerate(((rows0, sem0), (rows1, sem1))):
            rr = r + k
            gather(rr, buf, sem).wait()

            def acc_body(j, accs, buf=buf):
                accs = list(accs)
                row = j * _UNROLL
                for u in range(_UNROLL):
                    for c in range(_CHUNKS):
                        a = (u % 2) * _CHUNKS + c
                        accs[a] = accs[a] + buf[row + u, pl.ds(c * _LANES, _LANES)]
                return tuple(accs)

            zero = jnp.zeros((_LANES,), jnp.float32)
            accs = lax.fori_loop(0, _L // _UNROLL, acc_body, (zero,) * (2 * _CHUNKS))
            accs = [accs[c] + accs[_CHUNKS + c] for c in range(_CHUNKS)]

            @pl.when(rr + 2 < _RPW)
            def _(buf=buf, sem=sem, rr=rr):
                gather(rr + 2, buf, sem).start()

            for c in range(_CHUNKS):
                out_v[rr, pl.ds(c * _LANES, _LANES)] = jnp.maximum(
                    accs[c] + bias[c], 0.0
                )

    pltpu.sync_copy(out_v, out_hbm.at[pl.ds(base, _RPW)])


_TCV = 1024
_SPLIT = 49 * _TCV


def _linearize_body(lo_ref, hi_ref, out_ref):
    lo = jnp.transpose(lo_ref[...], (1, 0))
    hi = jnp.transpose(hi_ref[...], (1, 0))
    out_ref[...] = jnp.concatenate([lo, hi], axis=1)


def _linearize(Wt):
    return pl.pallas_call(
        _linearize_body,
        out_shape=jax.ShapeDtypeStruct((_SPLIT, 2 * _OUT), jnp.float32),
        grid=(_SPLIT // _TCV,),
        in_specs=[
            pl.BlockSpec((_OUT, _TCV), lambda i: (0, i)),
            pl.BlockSpec((_OUT, _TCV), lambda i: (0, i + _SPLIT // _TCV)),
        ],
        out_specs=pl.BlockSpec((_TCV, 2 * _OUT), lambda i: (i, 0)),
    )(Wt, Wt)


@jax.jit
def kernel(tokens, W, b):
    tok = tokens.astype(jnp.int32)
    tok = jnp.where(tok < _SPLIT, 2 * tok, 2 * (tok - _SPLIT) + 1)
    w_lin = _linearize(W.T).reshape(2 * _SPLIT, _OUT)
    run = functools.partial(
        pl.kernel,
        out_type=jax.ShapeDtypeStruct((_B, _OUT), jnp.float32),
        mesh=plsc.VectorSubcoreMesh(core_axis_name="c", subcore_axis_name="s"),
        scratch_types=[
            pltpu.VMEM((_RPW, _L), jnp.int32),
            pltpu.VMEM((_L, _OUT), jnp.float32),
            pltpu.VMEM((_L, _OUT), jnp.float32),
            pltpu.VMEM((_OUT,), jnp.float32),
            pltpu.VMEM((_RPW, _OUT), jnp.float32),
            pltpu.SemaphoreType.DMA,
            pltpu.SemaphoreType.DMA,
        ],
        compiler_params=pltpu.CompilerParams(use_tc_tiling_on_sc=False),
    )(_bow_body)
    return run(tok, w_lin, b)


# speedup vs baseline: 2.9308x; 1.0692x over previous
import functools

import jax
import jax.numpy as jnp
from jax import lax
from jax.experimental import pallas as pl
from jax.experimental.pallas import tpu as pltpu
from jax.experimental.pallas import tpu_sc as plsc

_VOCAB = 100000
_OUT = 64
_B = 1024
_L = 200

_NC = 2
_NS = 16
_NW = _NC * _NS
_RPW = _B // _NW
_LANES = 16
_CHUNKS = _OUT // _LANES
_UNROLL = 8


def _bow_body(
    tok_hbm, w_hbm, b_hbm, out_hbm, idx_v, rows0, rows1, b_v, out_v, sem0, sem1
):
    wid = lax.axis_index("s") * _NC + lax.axis_index("c")
    base = wid * _RPW

    pltpu.sync_copy(tok_hbm.at[pl.ds(base, _RPW)], idx_v)
    pltpu.sync_copy(b_hbm, b_v)

    bias = [b_v[pl.ds(c * _LANES, _LANES)] for c in range(_CHUNKS)]

    def gather(r, buf, sem):
        return pltpu.make_async_copy(
            w_hbm.at[idx_v.at[r]], buf, sem
        )

    gather(0, rows0, sem0).start()
    gather(1, rows1, sem1).start()

    @pl.loop(0, _RPW, step=2)
    def _(r):
        for k, (buf, sem) in enumerate(((rows0, sem0), (rows1, sem1))):
            rr = r + k
            gather(rr, buf, sem).wait()

            def acc_body(j, accs, buf=buf):
                accs = list(accs)
                row = j * _UNROLL
                for u in range(_UNROLL):
                    for c in range(_CHUNKS):
                        a = (u % 2) * _CHUNKS + c
                        accs[a] = accs[a] + buf[row + u, pl.ds(c * _LANES, _LANES)]
                return tuple(accs)

            zero = jnp.zeros((_LANES,), jnp.float32)
            accs = lax.fori_loop(0, _L // _UNROLL, acc_body, (zero,) * (2 * _CHUNKS))
            accs = [accs[c] + accs[_CHUNKS + c] for c in range(_CHUNKS)]

            @pl.when(rr + 2 < _RPW)
            def _(buf=buf, sem=sem, rr=rr):
                gather(rr + 2, buf, sem).start()

            for c in range(_CHUNKS):
                out_v[rr, pl.ds(c * _LANES, _LANES)] = jnp.maximum(
                    accs[c] + bias[c], 0.0
                )

    pltpu.sync_copy(out_v, out_hbm.at[pl.ds(base, _RPW)])


_TCV = 1024
_SPLIT = 49 * _TCV


def _linearize_body(lo_ref, hi_ref, out_ref):
    z = jnp.concatenate([lo_ref[...], hi_ref[...]], axis=0)
    out_ref[...] = jnp.transpose(z, (1, 0))


def _linearize(Wt):
    return pl.pallas_call(
        _linearize_body,
        out_shape=jax.ShapeDtypeStruct((_SPLIT, 2 * _OUT), jnp.float32),
        grid=(_SPLIT // _TCV,),
        in_specs=[
            pl.BlockSpec((_OUT, _TCV), lambda i: (0, i)),
            pl.BlockSpec((_OUT, _TCV), lambda i: (0, i + _SPLIT // _TCV)),
        ],
        out_specs=pl.BlockSpec((_TCV, 2 * _OUT), lambda i: (i, 0)),
    )(Wt, Wt)


@jax.jit
def kernel(tokens, W, b):
    tok = tokens.astype(jnp.int32)
    tok = jnp.where(tok < _SPLIT, 2 * tok, 2 * (tok - _SPLIT) + 1)
    w_lin = _linearize(W.T).reshape(2 * _SPLIT, _OUT)
    run = functools.partial(
        pl.kernel,
        out_type=jax.ShapeDtypeStruct((_B, _OUT), jnp.float32),
        mesh=plsc.VectorSubcoreMesh(core_axis_name="c", subcore_axis_name="s"),
        scratch_types=[
            pltpu.VMEM((_RPW, _L), jnp.int32),
            pltpu.VMEM((_L, _OUT), jnp.float32),
            pltpu.VMEM((_L, _OUT), jnp.float32),
            pltpu.VMEM((_OUT,), jnp.float32),
            pltpu.VMEM((_RPW, _OUT), jnp.float32),
            pltpu.SemaphoreType.DMA,
            pltpu.SemaphoreType.DMA,
        ],
        compiler_params=pltpu.CompilerParams(use_tc_tiling_on_sc=False),
    )(_bow_body)
    return run(tok, w_lin, b)


# speedup vs baseline: 3.1487x; 1.0744x over previous
import functools

import jax
import jax.numpy as jnp
from jax import lax
from jax.experimental import pallas as pl
from jax.experimental.pallas import tpu as pltpu
from jax.experimental.pallas import tpu_sc as plsc

_VOCAB = 100000
_OUT = 64
_B = 1024
_L = 200

_NC = 2
_NS = 16
_NW = _NC * _NS
_RPW = _B // _NW
_LANES = 16
_CHUNKS = _OUT // _LANES
_UNROLL = 8


def _bow_body(
    tok_hbm, w_hbm, b_hbm, out_hbm, idx_v, rows0, rows1, b_v, out_v, sem0, sem1
):
    wid = lax.axis_index("s") * _NC + lax.axis_index("c")
    base = wid * _RPW

    pltpu.sync_copy(tok_hbm.at[pl.ds(base, _RPW)], idx_v)
    pltpu.sync_copy(b_hbm, b_v)

    bias = [b_v[pl.ds(c * _LANES, _LANES)] for c in range(_CHUNKS)]

    def gather(r, buf, sem):
        return pltpu.make_async_copy(
            w_hbm.at[idx_v.at[r]], buf, sem
        )

    gather(0, rows0, sem0).start()
    gather(1, rows1, sem1).start()

    @pl.loop(0, _RPW, step=2)
    def _(r):
        for k, (buf, sem) in enumerate(((rows0, sem0), (rows1, sem1))):
            rr = r + k
            gather(rr, buf, sem).wait()

            def acc_body(j, accs, buf=buf):
                accs = list(accs)
                row = j * _UNROLL
                for u in range(_UNROLL):
                    for c in range(_CHUNKS):
                        a = (u % 2) * _CHUNKS + c
                        accs[a] = accs[a] + buf[row + u, pl.ds(c * _LANES, _LANES)]
                return tuple(accs)

            zero = jnp.zeros((_LANES,), jnp.float32)
            accs = lax.fori_loop(0, _L // _UNROLL, acc_body, (zero,) * (2 * _CHUNKS))
            accs = [accs[c] + accs[_CHUNKS + c] for c in range(_CHUNKS)]

            @pl.when(rr + 2 < _RPW)
            def _(buf=buf, sem=sem, rr=rr):
                gather(rr + 2, buf, sem).start()

            for c in range(_CHUNKS):
                out_v[rr, pl.ds(c * _LANES, _LANES)] = jnp.maximum(
                    accs[c] + bias[c], 0.0
                )

    pltpu.sync_copy(out_v, out_hbm.at[pl.ds(base, _RPW)])


_TCV = 1024
_SPLIT = 49 * _TCV


def _linearize_body(lo_ref, hi_ref, out_ref):
    i = pl.program_id(0)
    lo = lo_ref[:, pl.ds(i * _TCV, _TCV)]
    hi = hi_ref[:, pl.ds(i * _TCV, _TCV)]
    z = jnp.concatenate([lo, hi], axis=0)
    out_ref[...] = jnp.transpose(z, (1, 0))


def _linearize(Wt):
    return pl.pallas_call(
        _linearize_body,
        out_shape=jax.ShapeDtypeStruct((_SPLIT, 2 * _OUT), jnp.float32),
        grid=(_SPLIT // _TCV,),
        in_specs=[
            pl.BlockSpec((_OUT, _SPLIT), lambda i: (0, 0)),
            pl.BlockSpec((_OUT, _SPLIT), lambda i: (0, 1)),
        ],
        out_specs=pl.BlockSpec((_TCV, 2 * _OUT), lambda i: (i, 0)),
    )(Wt, Wt)


@jax.jit
def kernel(tokens, W, b):
    tok = tokens.astype(jnp.int32)
    tok = jnp.where(tok < _SPLIT, 2 * tok, 2 * (tok - _SPLIT) + 1)
    w_lin = _linearize(W.T).reshape(2 * _SPLIT, _OUT)
    run = functools.partial(
        pl.kernel,
        out_type=jax.ShapeDtypeStruct((_B, _OUT), jnp.float32),
        mesh=plsc.VectorSubcoreMesh(core_axis_name="c", subcore_axis_name="s"),
        scratch_types=[
            pltpu.VMEM((_RPW, _L), jnp.int32),
            pltpu.VMEM((_L, _OUT), jnp.float32),
            pltpu.VMEM((_L, _OUT), jnp.float32),
            pltpu.VMEM((_OUT,), jnp.float32),
            pltpu.VMEM((_RPW, _OUT), jnp.float32),
            pltpu.SemaphoreType.DMA,
            pltpu.SemaphoreType.DMA,
        ],
        compiler_params=pltpu.CompilerParams(use_tc_tiling_on_sc=False),
    )(_bow_body)
    return run(tok, w_lin, b)


# speedup vs baseline: 3.9085x; 1.2413x over previous
import functools

import jax
import jax.numpy as jnp
from jax import lax
from jax.experimental import pallas as pl
from jax.experimental.pallas import tpu as pltpu
from jax.experimental.pallas import tpu_sc as plsc

_VOCAB = 100000
_OUT = 64
_B = 1024
_L = 200

_NC = 2
_NS = 16
_NW = _NC * _NS
_RPW = _B // _NW
_LANES = 16
_CHUNKS = _OUT // _LANES
_UNROLL = 8


def _bow_body(
    tok_hbm, w_hbm, b_hbm, out_hbm, idx_v,
    rows0, rows1, rows2, rows3, b_v, out_v, sem0, sem1, sem2, sem3
):
    wid = lax.axis_index("s") * _NC + lax.axis_index("c")
    base = wid * _RPW

    pltpu.sync_copy(tok_hbm.at[pl.ds(base, _RPW)], idx_v)
    pltpu.sync_copy(b_hbm, b_v)

    bias = [b_v[pl.ds(c * _LANES, _LANES)] for c in range(_CHUNKS)]

    def gather(r, buf, sem):
        return pltpu.make_async_copy(
            w_hbm.at[idx_v.at[r]], buf, sem
        )

    bufs = ((rows0, sem0), (rows1, sem1), (rows2, sem2), (rows3, sem3))
    for k, (buf, sem) in enumerate(bufs):
        gather(k, buf, sem).start()

    @pl.loop(0, _RPW, step=len(bufs))
    def _(r):
        for k, (buf, sem) in enumerate(bufs):
            rr = r + k
            gather(rr, buf, sem).wait()

            def acc_body(j, accs, buf=buf):
                accs = list(accs)
                row = j * _UNROLL
                for u in range(_UNROLL):
                    for c in range(_CHUNKS):
                        a = (u % 2) * _CHUNKS + c
                        accs[a] = accs[a] + buf[row + u, pl.ds(c * _LANES, _LANES)]
                return tuple(accs)

            zero = jnp.zeros((_LANES,), jnp.float32)
            accs = lax.fori_loop(0, _L // _UNROLL, acc_body, (zero,) * (2 * _CHUNKS))
            accs = [accs[c] + accs[_CHUNKS + c] for c in range(_CHUNKS)]

            @pl.when(rr + len(bufs) < _RPW)
            def _(buf=buf, sem=sem, rr=rr):
                gather(rr + len(bufs), buf, sem).start()

            for c in range(_CHUNKS):
                out_v[rr, pl.ds(c * _LANES, _LANES)] = jnp.maximum(
                    accs[c] + bias[c], 0.0
                )

    pltpu.sync_copy(out_v, out_hbm.at[pl.ds(base, _RPW)])


_TCV = 2048
_SPLIT = 25 * _TCV


def _linearize_body(lo_ref, hi_ref, out_ref):
    i = pl.program_id(0)
    lo = lo_ref[:, pl.ds(i * _TCV, _TCV)]
    hi = hi_ref[:, pl.ds(i * _TCV, _TCV)]
    z = jnp.concatenate([lo, hi], axis=0)
    out_ref[...] = jnp.transpose(z, (1, 0))


def _linearize(Wt):
    return pl.pallas_call(
        _linearize_body,
        out_shape=jax.ShapeDtypeStruct((_SPLIT, 2 * _OUT), jnp.float32),
        grid=(_SPLIT // _TCV,),
        in_specs=[
            pl.BlockSpec((_OUT, _SPLIT), lambda i: (0, 0)),
            pl.BlockSpec((_OUT, _SPLIT), lambda i: (0, 1)),
        ],
        out_specs=pl.BlockSpec((_TCV, 2 * _OUT), lambda i: (i, 0)),
    )(Wt, Wt)


@jax.jit
def kernel(tokens, W, b):
    tok = tokens.astype(jnp.int32)
    tok = jnp.where(tok < _SPLIT, 2 * tok, 2 * (tok - _SPLIT) + 1)
    w_lin = _linearize(W.T).reshape(2 * _SPLIT, _OUT)
    run = functools.partial(
        pl.kernel,
        out_type=jax.ShapeDtypeStruct((_B, _OUT), jnp.float32),
        mesh=plsc.VectorSubcoreMesh(core_axis_name="c", subcore_axis_name="s"),
        scratch_types=[
            pltpu.VMEM((_RPW, _L), jnp.int32),
            pltpu.VMEM((_L, _OUT), jnp.float32),
            pltpu.VMEM((_L, _OUT), jnp.float32),
            pltpu.VMEM((_L, _OUT), jnp.float32),
            pltpu.VMEM((_L, _OUT), jnp.float32),
            pltpu.VMEM((_OUT,), jnp.float32),
            pltpu.VMEM((_RPW, _OUT), jnp.float32),
            pltpu.SemaphoreType.DMA,
            pltpu.SemaphoreType.DMA,
            pltpu.SemaphoreType.DMA,
            pltpu.SemaphoreType.DMA,
        ],
        compiler_params=pltpu.CompilerParams(use_tc_tiling_on_sc=False),
    )(_bow_body)
    return run(tok, w_lin, b)


# speedup vs baseline: 4.0716x; 1.0417x over previous
import functools

import jax
import jax.numpy as jnp
from jax import lax
from jax.experimental import pallas as pl
from jax.experimental.pallas import tpu as pltpu
from jax.experimental.pallas import tpu_sc as plsc

_VOCAB = 100000
_OUT = 64
_B = 1024
_L = 200

_NC = 2
_NS = 16
_NW = _NC * _NS
_RPW = _B // _NW
_LANES = 16
_CHUNKS = _OUT // _LANES
_UNROLL = 8


_NBUF = 8


def _bow_body(tok_hbm, w_hbm, b_hbm, out_hbm, idx_v, *scratch):
    rows_bufs = scratch[:_NBUF]
    b_v, out_v = scratch[_NBUF], scratch[_NBUF + 1]
    sems = scratch[_NBUF + 2:]
    wid = lax.axis_index("s") * _NC + lax.axis_index("c")
    base = wid * _RPW

    pltpu.sync_copy(tok_hbm.at[pl.ds(base, _RPW)], idx_v)
    pltpu.sync_copy(b_hbm, b_v)

    bias = [b_v[pl.ds(c * _LANES, _LANES)] for c in range(_CHUNKS)]

    def gather(r, buf, sem):
        return pltpu.make_async_copy(
            w_hbm.at[idx_v.at[r]], buf, sem
        )

    bufs = tuple(zip(rows_bufs, sems))
    for k, (buf, sem) in enumerate(bufs):
        gather(k, buf, sem).start()

    @pl.loop(0, _RPW, step=len(bufs))
    def _(r):
        for k, (buf, sem) in enumerate(bufs):
            rr = r + k
            gather(rr, buf, sem).wait()

            def acc_body(j, accs, buf=buf):
                accs = list(accs)
                row = j * _UNROLL
                for u in range(_UNROLL):
                    for c in range(_CHUNKS):
                        a = (u % 2) * _CHUNKS + c
                        accs[a] = accs[a] + buf[row + u, pl.ds(c * _LANES, _LANES)]
                return tuple(accs)

            zero = jnp.zeros((_LANES,), jnp.float32)
            accs = lax.fori_loop(0, _L // _UNROLL, acc_body, (zero,) * (2 * _CHUNKS))
            accs = [accs[c] + accs[_CHUNKS + c] for c in range(_CHUNKS)]

            @pl.when(rr + len(bufs) < _RPW)
            def _(buf=buf, sem=sem, rr=rr):
                gather(rr + len(bufs), buf, sem).start()

            for c in range(_CHUNKS):
                out_v[rr, pl.ds(c * _LANES, _LANES)] = jnp.maximum(
                    accs[c] + bias[c], 0.0
                )

    pltpu.sync_copy(out_v, out_hbm.at[pl.ds(base, _RPW)])


_TCV = 4096
_SPLIT = 13 * _TCV


def _linearize_body(lo_ref, hi_ref, out_ref):
    i = pl.program_id(0)
    lo = lo_ref[:, pl.ds(i * _TCV, _TCV)]
    hi = hi_ref[:, pl.ds(i * _TCV, _TCV)]
    z = jnp.concatenate([lo, hi], axis=0)
    out_ref[...] = jnp.transpose(z, (1, 0))


def _linearize(Wt):
    return pl.pallas_call(
        _linearize_body,
        out_shape=jax.ShapeDtypeStruct((_SPLIT, 2 * _OUT), jnp.float32),
        grid=(_SPLIT // _TCV,),
        in_specs=[
            pl.BlockSpec((_OUT, _SPLIT), lambda i: (0, 0)),
            pl.BlockSpec((_OUT, _SPLIT), lambda i: (0, 1)),
        ],
        out_specs=pl.BlockSpec((_TCV, 2 * _OUT), lambda i: (i, 0)),
    )(Wt, Wt)


@jax.jit
def kernel(tokens, W, b):
    tok = tokens.astype(jnp.int32)
    tok = jnp.where(tok < _SPLIT, 2 * tok, 2 * (tok - _SPLIT) + 1)
    w_lin = _linearize(W.T).reshape(2 * _SPLIT, _OUT)
    run = functools.partial(
        pl.kernel,
        out_type=jax.ShapeDtypeStruct((_B, _OUT), jnp.float32),
        mesh=plsc.VectorSubcoreMesh(core_axis_name="c", subcore_axis_name="s"),
        scratch_types=[
            pltpu.VMEM((_RPW, _L), jnp.int32),
            *[pltpu.VMEM((_L, _OUT), jnp.float32) for _ in range(_NBUF)],
            pltpu.VMEM((_OUT,), jnp.float32),
            pltpu.VMEM((_RPW, _OUT), jnp.float32),
            *[pltpu.SemaphoreType.DMA for _ in range(_NBUF)],
        ],
        compiler_params=pltpu.CompilerParams(use_tc_tiling_on_sc=False),
    )(_bow_body)
    return run(tok, w_lin, b)
